# baseline (speedup 1.0000x reference)
import jax
import jax.numpy as jnp
import numpy as np
from jax import lax
from jax.experimental import pallas as pl
from jax.experimental.pallas import tpu as pltpu
from jax.experimental.pallas import tpu_sc as plsc

VOCAB = 100000
D_MODEL = 1024
MAX_SEQ = 2048
PAD_IDX = 1
BATCH = 4
SEQ = 2048

NC = 2
NS = 16
NW = NC * NS
SEQ_PER_W = SEQ // NW
SUB = 8
N_STEPS = SEQ_PER_W // SUB
ROWS = BATCH * SUB
NBUF = 3
LANES = 16
NCH = D_MODEL // LANES


def _make_pos_enc() -> np.ndarray:
    pos = np.arange(MAX_SEQ, dtype=np.float32)[:, None]
    i = np.arange(0, d_model := D_MODEL, 2, dtype=np.float32)
    div = np.power(10000.0, i / float(d_model))
    pe = np.zeros((MAX_SEQ, d_model), dtype=np.float32)
    pe[:, 0::2] = np.sin(pos / div)
    pe[:, 1::2] = np.cos(pos / div)
    return pe


_PE = _make_pos_enc()[:SEQ]


def _embed_body(xt_hbm, tbl_hbm, pe_hbm, out_hbm,
                idx_v, scale_v, tok_v, pe_v,
                gsems, osems, psems):
    cid = lax.axis_index("c")
    sid = lax.axis_index("s")
    wid = sid * NC + cid
    s0 = wid * SEQ_PER_W

    pltpu.sync_copy(xt_hbm.at[wid], idx_v)

    def issue_g(k):
        return pltpu.async_copy(
            tbl_hbm.at[idx_v.at[k]], tok_v.at[k % NBUF], gsems.at[k % NBUF])

    def issue_pe(k):
        return pltpu.async_copy(
            pe_hbm.at[pl.ds(s0 + k * SUB, SUB)], pe_v.at[k % 2],
            psems.at[k % 2])

    def issue_o(k):
        return [pltpu.async_copy(
            tok_v.at[k % NBUF, pl.ds(b * SUB, SUB)],
            out_hbm.at[b, wid, pl.ds(k * SUB, SUB)], osems.at[k % NBUF])
            for b in range(BATCH)]

    g_cp = {0: issue_g(0), 1: issue_g(1)}
    p_cp = {0: issue_pe(0), 1: issue_pe(1)}
    o_cp = {}

    padcnt = jnp.int32(0)
    for k in range(N_STEPS):
        for g in range(ROWS // LANES):
            iv = idx_v[k, pl.ds(g * LANES, LANES)]
            hit = jnp.where(iv == PAD_IDX, 1, 0).astype(jnp.int32)
            scale_v[pl.ds(k * ROWS + g * LANES, LANES)] = (
                1.0 - hit.astype(jnp.float32))
            padcnt = padcnt + jnp.sum(hit)

    for k in range(N_STEPS):
        kb = k % NBUF
        kp = k % 2
        g_cp[k].wait()
        p_cp[k].wait()

        @pl.when(padcnt > 0)
        def _scale_pass(k=k, kb=kb):
            def srow(r, carry):
                b = r // SUB
                sp = r % SUB
                rid = jnp.broadcast_to(k * ROWS + r, (LANES,)).astype(
                    jnp.int32)
                scale = plsc.load_gather(scale_v, [rid])
                for j in range(NCH):
                    sl = pl.ds(j * LANES, LANES)
                    tok_v[kb, r, sl] = tok_v[kb, r, sl] * scale
                return carry
            lax.fori_loop(0, ROWS, srow, 0)

        def arow(sp, carry, kb=kb, kp=kp):
            for j in range(NCH):
                sl = pl.ds(j * LANES, LANES)
                pvec = pe_v[kp, sp, sl]
                for b in range(BATCH):
                    plsc.addupdate(tok_v.at[kb, b * SUB + sp, sl], pvec)
            return carry

        lax.fori_loop(0, SUB, arow, 0)
        o_cp[k] = issue_o(k)
        if k + 2 < N_STEPS:
            if k >= 1:
                for d in o_cp[k - 1]:
                    d.wait()
            g_cp[k + 2] = issue_g(k + 2)
            p_cp[k + 2] = issue_pe(k + 2)

    for k in range(N_STEPS - 3, N_STEPS):
        for d in o_cp[k]:
            d.wait()


@jax.jit
def kernel(x, token_table):
    pe = jnp.asarray(_PE)
    x_t = (x.reshape(BATCH, NW, N_STEPS, SUB)
           .transpose(1, 2, 0, 3)
           .reshape(NW, N_STEPS, ROWS))
    mesh = plsc.VectorSubcoreMesh(core_axis_name="c", subcore_axis_name="s")
    out = pl.kernel(
        _embed_body,
        mesh=mesh,
        compiler_params=pltpu.CompilerParams(needs_layout_passes=False),
        out_type=jax.ShapeDtypeStruct((BATCH, NW, SEQ_PER_W, D_MODEL),
                                      jnp.float32),
        scratch_types=[
            pltpu.VMEM((N_STEPS, ROWS), jnp.int32),
            pltpu.VMEM((N_STEPS * ROWS,), jnp.float32),
            pltpu.VMEM((NBUF, ROWS, D_MODEL), jnp.float32),
            pltpu.VMEM((2, SUB, D_MODEL), jnp.float32),
            pltpu.SemaphoreType.DMA((NBUF,)),
            pltpu.SemaphoreType.DMA((NBUF,)),
            pltpu.SemaphoreType.DMA((2,)),
        ],
    )(x_t, token_table, pe)
    return out.reshape(BATCH, SEQ, D_MODEL)


# speedup vs baseline: 1.1018x; 1.1018x over previous
import jax
import jax.numpy as jnp
import numpy as np
from jax import lax
from jax.experimental import pallas as pl
from jax.experimental.pallas import tpu as pltpu
from jax.experimental.pallas import tpu_sc as plsc

VOCAB = 100000
D_MODEL = 1024
MAX_SEQ = 2048
PAD_IDX = 1
BATCH = 4
SEQ = 2048

NC = 2
NS = 16
NW = NC * NS
SEQ_PER_W = SEQ // NW
SUB = 4
N_STEPS = SEQ_PER_W // SUB
ROWS = BATCH * SUB
NBUF = 4
LANES = 16
NCH = D_MODEL // LANES


def _make_pos_enc() -> np.ndarray:
    pos = np.arange(MAX_SEQ, dtype=np.float32)[:, None]
    i = np.arange(0, d_model := D_MODEL, 2, dtype=np.float32)
    div = np.power(10000.0, i / float(d_model))
    pe = np.zeros((MAX_SEQ, d_model), dtype=np.float32)
    pe[:, 0::2] = np.sin(pos / div)
    pe[:, 1::2] = np.cos(pos / div)
    return pe


_PE = _make_pos_enc()[:SEQ]


def _embed_body(xt_hbm, tbl_hbm, pe_hbm, out_hbm,
                idx_v, scale_v, tok_v, pe_v,
                gsems, osems, psems):
    cid = lax.axis_index("c")
    sid = lax.axis_index("s")
    wid = sid * NC + cid
    s0 = wid * SEQ_PER_W

    pltpu.sync_copy(xt_hbm.at[wid], idx_v)

    def g_desc(k, kb):
        return pltpu.make_async_copy(
            tbl_hbm.at[idx_v.at[k]], tok_v.at[kb], gsems.at[kb])

    def pe_desc(k, kb):
        return pltpu.make_async_copy(
            pe_hbm.at[pl.ds(s0 + k * SUB, SUB)], pe_v.at[kb], psems.at[kb])

    def o_descs(k, kb):
        return [pltpu.make_async_copy(
            tok_v.at[kb, pl.ds(b * SUB, SUB)],
            out_hbm.at[b, wid, pl.ds(k * SUB, SUB)], osems.at[kb])
            for b in range(BATCH)]

    for k in range(NBUF - 1):
        g_desc(k, k).start()
        pe_desc(k, k).start()

    padcnt = jnp.int32(0)
    for k in range(N_STEPS):
        for g in range(ROWS // LANES):
            iv = idx_v[k, pl.ds(g * LANES, LANES)]
            hit = jnp.where(iv == PAD_IDX, 1, 0).astype(jnp.int32)
            scale_v[pl.ds(k * ROWS + g * LANES, LANES)] = (
                1.0 - hit.astype(jnp.float32))
            padcnt = padcnt + jnp.sum(hit)

    def step(k, carry):
        kb = lax.rem(k, NBUF)
        g_desc(k, kb).wait()
        pe_desc(k, kb).wait()

        @pl.when(padcnt > 0)
        def _scale_pass():
            def srow(r, carry2):
                rid = jnp.broadcast_to(k * ROWS + r, (LANES,)).astype(
                    jnp.int32)
                scale = plsc.load_gather(scale_v, [rid])
                for j in range(NCH):
                    sl = pl.ds(j * LANES, LANES)
                    tok_v[kb, r, sl] = tok_v[kb, r, sl] * scale
                return carry2
            lax.fori_loop(0, ROWS, srow, 0)

        def arow(sp, carry2):
            for j in range(NCH):
                sl = pl.ds(j * LANES, LANES)
                pvec = pe_v[kb, sp, sl]
                for b in range(BATCH):
                    plsc.addupdate(tok_v.at[kb, b * SUB + sp, sl], pvec)
            return carry2

        lax.fori_loop(0, SUB, arow, 0)
        for d in o_descs(k, kb):
            d.start()

        @pl.when(k + NBUF - 1 < N_STEPS)
        def _prefetch():
            kn = k + NBUF - 1
            knb = lax.rem(kn, NBUF)

            @pl.when(k >= 1)
            def _drain_prev():
                for d in o_descs(k - 1, knb):
                    d.wait()

            g_desc(kn, knb).start()
            pe_desc(kn, knb).start()

        return carry

    lax.fori_loop(0, N_STEPS, step, 0)

    for k in range(N_STEPS - NBUF, N_STEPS):
        for d in o_descs(k, k % NBUF):
            d.wait()


@jax.jit
def kernel(x, token_table):
    pe = jnp.asarray(_PE)
    x_t = (x.reshape(BATCH, NW, N_STEPS, SUB)
           .transpose(1, 2, 0, 3)
           .reshape(NW, N_STEPS, ROWS))
    mesh = plsc.VectorSubcoreMesh(core_axis_name="c", subcore_axis_name="s")
    out = pl.kernel(
        _embed_body,
        mesh=mesh,
        compiler_params=pltpu.CompilerParams(needs_layout_passes=False),
        out_type=jax.ShapeDtypeStruct((BATCH, NW, SEQ_PER_W, D_MODEL),
                                      jnp.float32),
        scratch_types=[
            pltpu.VMEM((N_STEPS, ROWS), jnp.int32),
            pltpu.VMEM((N_STEPS * ROWS,), jnp.float32),
            pltpu.VMEM((NBUF, ROWS, D_MODEL), jnp.float32),
            pltpu.VMEM((NBUF, SUB, D_MODEL), jnp.float32),
            pltpu.SemaphoreType.DMA((NBUF,)),
            pltpu.SemaphoreType.DMA((NBUF,)),
            pltpu.SemaphoreType.DMA((NBUF,)),
        ],
    )(x_t, token_table, pe)
    return out.reshape(BATCH, SEQ, D_MODEL)


# speedup vs baseline: 1.1266x; 1.0225x over previous
import jax
import jax.numpy as jnp
import numpy as np
from jax import lax
from jax.experimental import pallas as pl
from jax.experimental.pallas import tpu as pltpu
from jax.experimental.pallas import tpu_sc as plsc

VOCAB = 100000
D_MODEL = 1024
MAX_SEQ = 2048
PAD_IDX = 1
BATCH = 4
SEQ = 2048

NC = 2
NS = 16
NW = NC * NS
SEQ_PER_W = SEQ // NW
SUB = 4
N_STEPS = SEQ_PER_W // SUB
ROWS = BATCH * SUB
NBUF = 6
LANES = 16
NCH = D_MODEL // LANES


def _make_pos_enc() -> np.ndarray:
    pos = np.arange(MAX_SEQ, dtype=np.float32)[:, None]
    i = np.arange(0, d_model := D_MODEL, 2, dtype=np.float32)
    div = np.power(10000.0, i / float(d_model))
    pe = np.zeros((MAX_SEQ, d_model), dtype=np.float32)
    pe[:, 0::2] = np.sin(pos / div)
    pe[:, 1::2] = np.cos(pos / div)
    return pe


_PE = _make_pos_enc()[:SEQ]


def _embed_body(xt_hbm, tbl_hbm, pe_hbm, out_hbm,
                idx_v, scale_v, tok_v, pe_v,
                gsems, osems, psems):
    cid = lax.axis_index("c")
    sid = lax.axis_index("s")
    wid = sid * NC + cid
    s0 = wid * SEQ_PER_W

    pltpu.sync_copy(xt_hbm.at[wid], idx_v)

    def g_desc(k, kb):
        return pltpu.make_async_copy(
            tbl_hbm.at[idx_v.at[k]], tok_v.at[kb], gsems.at[kb])

    def pe_desc(k, kb):
        return pltpu.make_async_copy(
            pe_hbm.at[pl.ds(s0 + k * SUB, SUB)], pe_v.at[kb], psems.at[kb])

    def o_descs(k, kb):
        return [pltpu.make_async_copy(
            tok_v.at[kb, pl.ds(b * SUB, SUB)],
            out_hbm.at[b, wid, pl.ds(k * SUB, SUB)], osems.at[kb])
            for b in range(BATCH)]

    for k in range(NBUF - 1):
        g_desc(k, k).start()
        pe_desc(k, k).start()

    padcnt = jnp.int32(0)
    for k in range(N_STEPS):
        for g in range(ROWS // LANES):
            iv = idx_v[k, pl.ds(g * LANES, LANES)]
            hit = jnp.where(iv == PAD_IDX, 1, 0).astype(jnp.int32)
            scale_v[pl.ds(k * ROWS + g * LANES, LANES)] = (
                1.0 - hit.astype(jnp.float32))
            padcnt = padcnt + jnp.sum(hit)

    def step(k, carry):
        kb = lax.rem(k, NBUF)
        g_desc(k, kb).wait()
        pe_desc(k, kb).wait()

        @pl.when(padcnt > 0)
        def _scale_pass():
            def srow(r, carry2):
                rid = jnp.broadcast_to(k * ROWS + r, (LANES,)).astype(
                    jnp.int32)
                scale = plsc.load_gather(scale_v, [rid])
                for j in range(NCH):
                    sl = pl.ds(j * LANES, LANES)
                    tok_v[kb, r, sl] = tok_v[kb, r, sl] * scale
                return carry2
            lax.fori_loop(0, ROWS, srow, 0)

        def arow(sp, carry2):
            for j in range(NCH):
                sl = pl.ds(j * LANES, LANES)
                pvec = pe_v[kb, sp, sl]
                for b in range(BATCH):
                    plsc.addupdate(tok_v.at[kb, b * SUB + sp, sl], pvec)
            return carry2

        lax.fori_loop(0, SUB, arow, 0)
        for d in o_descs(k, kb):
            d.start()

        @pl.when(k + NBUF - 1 < N_STEPS)
        def _prefetch():
            kn = k + NBUF - 1
            knb = lax.rem(kn, NBUF)

            @pl.when(k >= 1)
            def _drain_prev():
                for d in o_descs(k - 1, knb):
                    d.wait()

            g_desc(kn, knb).start()
            pe_desc(kn, knb).start()

        return carry

    lax.fori_loop(0, N_STEPS, step, 0)

    for k in range(N_STEPS - NBUF, N_STEPS):
        for d in o_descs(k, k % NBUF):
            d.wait()


@jax.jit
def kernel(x, token_table):
    pe = jnp.asarray(_PE)
    x_t = (x.reshape(BATCH, NW, N_STEPS, SUB)
           .transpose(1, 2, 0, 3)
           .reshape(NW, N_STEPS, ROWS))
    mesh = plsc.VectorSubcoreMesh(core_axis_name="c", subcore_axis_name="s")
    out = pl.kernel(
        _embed_body,
        mesh=mesh,
        compiler_params=pltpu.CompilerParams(needs_layout_passes=False),
        out_type=jax.ShapeDtypeStruct((BATCH, NW, SEQ_PER_W, D_MODEL),
                                      jnp.float32),
        scratch_types=[
            pltpu.VMEM((N_STEPS, ROWS), jnp.int32),
            pltpu.VMEM((N_STEPS * ROWS,), jnp.float32),
            pltpu.VMEM((NBUF, ROWS, D_MODEL), jnp.float32),
            pltpu.VMEM((NBUF, SUB, D_MODEL), jnp.float32),
            pltpu.SemaphoreType.DMA((NBUF,)),
            pltpu.SemaphoreType.DMA((NBUF,)),
            pltpu.SemaphoreType.DMA((NBUF,)),
        ],
    )(x_t, token_table, pe)
    return out.reshape(BATCH, SEQ, D_MODEL)
